# baseline (speedup 1.0000x reference)
import dataclasses

import jax
import jax.numpy as jnp
from jax import lax
from jax.experimental import pallas as pl
from jax.experimental.pallas import tpu as pltpu
from jax.experimental.pallas import tpu_sc as plsc

N = 10000
E = 320000
IN = 128
HID = 256
OUT = 128

NC = 2
NS = 16
NW = NC * NS
B = 40
C1 = E // NW // B
RPT = N // NS
RB = 2000
NBLK = N // RB
NBUF = 5

_MESH = dict(mesh=plsc.VectorSubcoreMesh(core_axis_name="c", subcore_axis_name="s"))
_f32 = jnp.float32

_NO_LAYOUT_CP = pltpu.CompilerParams()
if "needs_layout_passes" in pltpu.CompilerParams.__dataclass_fields__:
    _NO_LAYOUT_CP = dataclasses.replace(_NO_LAYOUT_CP, needs_layout_passes=False)


BD = 80
CD = E // NW // BD


def _deg_body(dst_hbm, out_hbm, idx_v, hist_v):
    c = lax.axis_index("c")
    s = lax.axis_index("s")
    w = c * NS + s
    pltpu.sync_copy(dst_hbm.at[w], idx_v)

    @pl.loop(0, N // 16)
    def _(i):
        hist_v[pl.ds(i * 16, 16)] = jnp.zeros((16,), _f32)

    ones = jnp.ones((16,), _f32)

    @pl.loop(0, CD)
    def _(j):
        for k in range(BD // 16):
            v = idx_v[j, pl.ds(k * 16, 16)]
            plsc.addupdate_scatter(hist_v, [v], ones)

    pltpu.sync_copy(hist_v, out_hbm.at[w])


def _deg_call(dst1):
    return pl.kernel(
        _deg_body,
        out_type=jax.ShapeDtypeStruct((NW, N), _f32),
        scratch_types=[
            pltpu.VMEM((CD, BD), jnp.int32),
            pltpu.VMEM((N,), _f32),
        ],
        compiler_params=_NO_LAYOUT_CP,
        **_MESH,
    )(dst1)


NSEG = 5
CSEG = C1 // NSEG
NGRP = CSEG // NBUF


def _agg_body(tab_hbm, src_hbm, dst_hbm, z_hbm, out_hbm, srcv, dstv, rows,
              acc_sh):
    c = lax.axis_index("c")
    s = lax.axis_index("s")
    w = c * NS + s
    pltpu.sync_copy(z_hbm, acc_sh.at[pl.ds(s * RPT, RPT)])
    plsc.subcore_barrier()

    def inner(gsem, ssem):
        def wait_gather(b, j):
            pltpu.make_async_copy(tab_hbm.at[srcv.at[j]], rows[b],
                                  gsem[b]).wait()

        def wait_scatter(b, j):
            pltpu.make_async_copy(rows[b], acc_sh.at[dstv.at[j]],
                                  ssem[b]).wait()

        @pl.loop(0, NSEG)
        def _(seg):
            pltpu.sync_copy(src_hbm.at[w, seg], srcv)
            pltpu.sync_copy(dst_hbm.at[w, seg], dstv)
            for b in range(NBUF):
                pltpu.async_copy(tab_hbm.at[srcv.at[b]], rows[b], gsem[b])

            @pl.loop(0, NGRP - 1)
            def _(g):
                j0 = g * NBUF
                for b in range(NBUF):
                    wait_gather(b, j0 + b)
                    pltpu.async_copy(rows[b], acc_sh.at[dstv.at[j0 + b]],
                                     ssem[b], add=True)
                for b in range(NBUF):
                    wait_scatter(b, j0 + b)
                    pltpu.async_copy(tab_hbm.at[srcv.at[j0 + NBUF + b]],
                                     rows[b], gsem[b])

            j0 = (NGRP - 1) * NBUF
            for b in range(NBUF):
                wait_gather(b, j0 + b)
                pltpu.async_copy(rows[b], acc_sh.at[dstv.at[j0 + b]],
                                 ssem[b], add=True)
            for b in range(NBUF):
                wait_scatter(b, j0 + b)

    pl.run_scoped(
        inner,
        gsem=[pltpu.SemaphoreType.DMA for _ in range(NBUF)],
        ssem=[pltpu.SemaphoreType.DMA for _ in range(NBUF)],
    )

    plsc.subcore_barrier()
    pltpu.sync_copy(acc_sh.at[pl.ds(s * RPT, RPT)], out_hbm.at[w])


_AGG_KERNEL = pl.kernel(
    _agg_body,
    out_type=jax.ShapeDtypeStruct((NW, RPT, IN), _f32),
    scratch_types=[
        pltpu.VMEM((CSEG, B), jnp.int32),
        pltpu.VMEM((CSEG, B), jnp.int32),
        [pltpu.VMEM((B, IN), _f32) for _ in range(NBUF)],
        pltpu.VMEM_SHARED((N, IN), _f32),
    ],
    **_MESH,
)


def _agg_call(tab, src1, dst1, z128):
    return _AGG_KERNEL(tab, src1, dst1, z128).reshape(NC * N, IN)


def _tc1_body(degt_ref, x_ref, dinv_ref, u_ref):
    deg = jnp.sum(degt_ref[...], axis=1, keepdims=True) + 1.0
    dinv = lax.rsqrt(deg)
    dinv_ref[...] = dinv
    u_ref[...] = x_ref[...] * dinv


def _tc1_call(degt, x):
    return pl.pallas_call(
        _tc1_body,
        grid=(NBLK,),
        in_specs=[
            pl.BlockSpec((RB, NW), lambda i: (i, 0)),
            pl.BlockSpec((RB, IN), lambda i: (i, 0)),
        ],
        out_specs=[
            pl.BlockSpec((RB, 1), lambda i: (i, 0)),
            pl.BlockSpec((RB, IN), lambda i: (i, 0)),
        ],
        out_shape=[
            jax.ShapeDtypeStruct((N, 1), _f32),
            jax.ShapeDtypeStruct((NC * N, IN), _f32),
        ],
    )(degt, x)


def _dot(a, b):
    return jnp.dot(a, b, preferred_element_type=_f32,
                   precision=lax.Precision.HIGHEST)


def _tc2_body(s1a_ref, s1b_ref, u_ref, dinv_ref, w1_ref, b1_ref, uh_ref):
    dinv = dinv_ref[...]
    aggx = dinv * (s1a_ref[...] + s1b_ref[...] + u_ref[...])
    h = jnp.maximum(_dot(aggx, w1_ref[...]) + b1_ref[...], 0.0)
    uh_ref[...] = h * dinv


def _tc2_call(s1, u, dinv, W1, b1r):
    return pl.pallas_call(
        _tc2_body,
        grid=(NC, NBLK),
        in_specs=[
            pl.BlockSpec((RB, IN), lambda c, i: (i, 0)),
            pl.BlockSpec((RB, IN), lambda c, i: (NBLK + i, 0)),
            pl.BlockSpec((RB, IN), lambda c, i: (i, 0)),
            pl.BlockSpec((RB, 1), lambda c, i: (i, 0)),
            pl.BlockSpec((IN, HID // NC), lambda c, i: (0, c)),
            pl.BlockSpec((1, HID // NC), lambda c, i: (0, c)),
        ],
        out_specs=pl.BlockSpec((RB, HID // NC), lambda c, i: (c * NBLK + i, 0)),
        out_shape=jax.ShapeDtypeStruct((NC * N, HID // NC), _f32),
    )(s1, s1, u, dinv, W1, b1r)


def _tc3_body(s2a0_ref, s2a1_ref, s2b0_ref, s2b1_ref, uha_ref, uhb_ref,
              dinv_ref, wmu_ref, bmu_ref, wls_ref, bls_ref, mu_ref, ls_ref):
    dinv = dinv_ref[...]
    a0 = dinv * (s2a0_ref[...] + s2a1_ref[...] + uha_ref[...])
    a1 = dinv * (s2b0_ref[...] + s2b1_ref[...] + uhb_ref[...])
    mu_ref[...] = (_dot(a0, wmu_ref[0:128, :]) + _dot(a1, wmu_ref[128:256, :])
                   + bmu_ref[...])
    ls_ref[...] = (_dot(a0, wls_ref[0:128, :]) + _dot(a1, wls_ref[128:256, :])
                   + bls_ref[...])


def _tc3_call(s2a, s2b, uh, dinv, Wmu, bmur, Wls, blsr):
    half = lambda off: pl.BlockSpec((RB, OUT), lambda i, off=off: (off + i, 0))
    return pl.pallas_call(
        _tc3_body,
        grid=(NBLK,),
        in_specs=[
            half(0), half(NBLK),
            half(0), half(NBLK),
            half(0), half(NBLK),
            pl.BlockSpec((RB, 1), lambda i: (i, 0)),
            pl.BlockSpec((HID, OUT), lambda i: (0, 0)),
            pl.BlockSpec((1, OUT), lambda i: (0, 0)),
            pl.BlockSpec((HID, OUT), lambda i: (0, 0)),
            pl.BlockSpec((1, OUT), lambda i: (0, 0)),
        ],
        out_specs=[
            pl.BlockSpec((RB, OUT), lambda i: (i, 0)),
            pl.BlockSpec((RB, OUT), lambda i: (i, 0)),
        ],
        out_shape=[
            jax.ShapeDtypeStruct((N, OUT), _f32),
            jax.ShapeDtypeStruct((N, OUT), _f32),
        ],
    )(s2a, s2a, s2b, s2b, uh, uh, dinv, Wmu, bmur, Wls, blsr)


@jax.jit
def _run(x, edge_index, W1, b1, Wmu, bmu, Wls, bls):
    ei = edge_index.astype(jnp.int32)
    src, dst = ei[0], ei[1]
    dst1 = dst.reshape(NW, CD, BD)
    src5 = src.reshape(NW, NSEG, CSEG, B)
    dst5 = dst.reshape(NW, NSEG, CSEG, B)
    src_stack = jnp.stack([src5, src5, src5 + N])
    z128 = jnp.zeros((RPT, 128), _f32)
    b1r = b1.reshape(1, HID)

    degt = _deg_call(dst1).T
    dinv, u = _tc1_call(degt, x)

    n_iter = lax.optimization_barrier(jnp.int32(3))

    def cond_fn(st):
        return st[0] < n_iter

    def body_fn(st):
        i, tab, _, s_prev = st
        src_i = lax.dynamic_index_in_dim(src_stack, i, keepdims=False)
        s = _agg_call(tab, src_i, dst5, z128)
        new_tab = lax.cond(
            i == 0,
            lambda: _tc2_call(s, u, dinv, W1, b1r),
            lambda: tab,
        )
        return (i + 1, new_tab, s_prev, s)

    _, uh, s2a, s2b = lax.while_loop(
        cond_fn, body_fn, (jnp.int32(0), u, u, u))

    mu, ls = _tc3_call(s2a, s2b, uh, dinv, Wmu, bmu.reshape(1, OUT),
                       Wls, bls.reshape(1, OUT))
    return (mu, ls)


def kernel(x, edge_index, W1, b1, Wmu, bmu, Wls, bls):
    return _run(x, edge_index, W1, b1, Wmu, bmu, Wls, bls)


# speedup vs baseline: 1.0151x; 1.0151x over previous
import dataclasses

import jax
import jax.numpy as jnp
from jax import lax
from jax.experimental import pallas as pl
from jax.experimental.pallas import tpu as pltpu
from jax.experimental.pallas import tpu_sc as plsc

N = 10000
E = 320000
IN = 128
HID = 256
OUT = 128

NC = 2
NS = 16
NW = NC * NS
B = 40
C1 = E // NW // B
RPT = N // NS
RB = 2000
NBLK = N // RB
NBUF = 5

_MESH = dict(mesh=plsc.VectorSubcoreMesh(core_axis_name="c", subcore_axis_name="s"))
_f32 = jnp.float32

_NO_LAYOUT_CP = pltpu.CompilerParams()
if "needs_layout_passes" in pltpu.CompilerParams.__dataclass_fields__:
    _NO_LAYOUT_CP = dataclasses.replace(_NO_LAYOUT_CP, needs_layout_passes=False)


BD = 80
CD = E // NW // BD


def _deg_body(dst_hbm, out_hbm, idx_v, hist_v):
    c = lax.axis_index("c")
    s = lax.axis_index("s")
    w = c * NS + s
    pltpu.sync_copy(dst_hbm.at[w], idx_v)

    @pl.loop(0, N // 16)
    def _(i):
        hist_v[pl.ds(i * 16, 16)] = jnp.zeros((16,), _f32)

    ones = jnp.ones((16,), _f32)

    @pl.loop(0, CD)
    def _(j):
        for k in range(BD // 16):
            v = idx_v[j, pl.ds(k * 16, 16)]
            plsc.addupdate_scatter(hist_v, [v], ones)

    pltpu.sync_copy(hist_v, out_hbm.at[w])


def _deg_call(dst1):
    return pl.kernel(
        _deg_body,
        out_type=jax.ShapeDtypeStruct((NW, N), _f32),
        scratch_types=[
            pltpu.VMEM((CD, BD), jnp.int32),
            pltpu.VMEM((N,), _f32),
        ],
        compiler_params=_NO_LAYOUT_CP,
        **_MESH,
    )(dst1)


NSEG = 5
CSEG = C1 // NSEG
NGRP = CSEG // NBUF


def _agg_body(tab_hbm, sd_hbm, z_hbm, out_hbm, sdv, rows, acc_sh):
    c = lax.axis_index("c")
    s = lax.axis_index("s")
    w = c * NS + s
    srcv = sdv.at[0]
    dstv = sdv.at[1]
    pltpu.sync_copy(z_hbm, acc_sh.at[pl.ds(s * RPT, RPT)])
    plsc.subcore_barrier()

    def inner(gsem, ssem):
        def wait_gather(b, j):
            pltpu.make_async_copy(tab_hbm.at[srcv.at[j]], rows[b],
                                  gsem[b]).wait()

        def wait_scatter(b, j):
            pltpu.make_async_copy(rows[b], acc_sh.at[dstv.at[j]],
                                  ssem[b]).wait()

        @pl.loop(0, NSEG)
        def _(seg):
            pltpu.sync_copy(sd_hbm.at[w, seg], sdv)
            for b in range(NBUF):
                pltpu.async_copy(tab_hbm.at[srcv.at[b]], rows[b], gsem[b])

            @pl.loop(0, NGRP - 1)
            def _(g):
                j0 = g * NBUF
                for b in range(NBUF):
                    wait_gather(b, j0 + b)
                    pltpu.async_copy(rows[b], acc_sh.at[dstv.at[j0 + b]],
                                     ssem[b], add=True)
                for b in range(NBUF):
                    wait_scatter(b, j0 + b)
                    pltpu.async_copy(tab_hbm.at[srcv.at[j0 + NBUF + b]],
                                     rows[b], gsem[b])

            j0 = (NGRP - 1) * NBUF
            for b in range(NBUF):
                wait_gather(b, j0 + b)
                pltpu.async_copy(rows[b], acc_sh.at[dstv.at[j0 + b]],
                                 ssem[b], add=True)
            for b in range(NBUF):
                wait_scatter(b, j0 + b)

    pl.run_scoped(
        inner,
        gsem=[pltpu.SemaphoreType.DMA for _ in range(NBUF)],
        ssem=[pltpu.SemaphoreType.DMA for _ in range(NBUF)],
    )

    plsc.subcore_barrier()
    pltpu.sync_copy(acc_sh.at[pl.ds(s * RPT, RPT)], out_hbm.at[w])


_AGG_KERNEL = pl.kernel(
    _agg_body,
    out_type=jax.ShapeDtypeStruct((NW, RPT, IN), _f32),
    scratch_types=[
        pltpu.VMEM((2, CSEG, B), jnp.int32),
        [pltpu.VMEM((B, IN), _f32) for _ in range(NBUF)],
        pltpu.VMEM_SHARED((N, IN), _f32),
    ],
    **_MESH,
)


def _agg_call(tab, sd, z128):
    return _AGG_KERNEL(tab, sd, z128).reshape(NC * N, IN)


def _tc1_body(degt_ref, x_ref, dinv_ref, u_ref):
    deg = jnp.sum(degt_ref[...], axis=1, keepdims=True) + 1.0
    dinv = lax.rsqrt(deg)
    dinv_ref[...] = dinv
    u_ref[...] = x_ref[...] * dinv


def _tc1_call(degt, x):
    return pl.pallas_call(
        _tc1_body,
        grid=(NBLK,),
        in_specs=[
            pl.BlockSpec((RB, NW), lambda i: (i, 0)),
            pl.BlockSpec((RB, IN), lambda i: (i, 0)),
        ],
        out_specs=[
            pl.BlockSpec((RB, 1), lambda i: (i, 0)),
            pl.BlockSpec((RB, IN), lambda i: (i, 0)),
        ],
        out_shape=[
            jax.ShapeDtypeStruct((N, 1), _f32),
            jax.ShapeDtypeStruct((NC * N, IN), _f32),
        ],
    )(degt, x)


def _dot(a, b):
    return jnp.dot(a, b, preferred_element_type=_f32,
                   precision=lax.Precision.HIGHEST)


def _tc2_body(s1a_ref, s1b_ref, u_ref, dinv_ref, w1_ref, b1_ref, uh_ref):
    dinv = dinv_ref[...]
    aggx = dinv * (s1a_ref[...] + s1b_ref[...] + u_ref[...])
    h = jnp.maximum(_dot(aggx, w1_ref[...]) + b1_ref[...], 0.0)
    uh_ref[...] = h * dinv


def _tc2_call(s1, u, dinv, W1, b1r):
    return pl.pallas_call(
        _tc2_body,
        grid=(NC, NBLK),
        in_specs=[
            pl.BlockSpec((RB, IN), lambda c, i: (i, 0)),
            pl.BlockSpec((RB, IN), lambda c, i: (NBLK + i, 0)),
            pl.BlockSpec((RB, IN), lambda c, i: (i, 0)),
            pl.BlockSpec((RB, 1), lambda c, i: (i, 0)),
            pl.BlockSpec((IN, HID // NC), lambda c, i: (0, c)),
            pl.BlockSpec((1, HID // NC), lambda c, i: (0, c)),
        ],
        out_specs=pl.BlockSpec((RB, HID // NC), lambda c, i: (c * NBLK + i, 0)),
        out_shape=jax.ShapeDtypeStruct((NC * N, HID // NC), _f32),
    )(s1, s1, u, dinv, W1, b1r)


def _tc3_body(s2a0_ref, s2a1_ref, s2b0_ref, s2b1_ref, uha_ref, uhb_ref,
              dinv_ref, wmu_ref, bmu_ref, wls_ref, bls_ref, mu_ref, ls_ref):
    dinv = dinv_ref[...]
    a0 = dinv * (s2a0_ref[...] + s2a1_ref[...] + uha_ref[...])
    a1 = dinv * (s2b0_ref[...] + s2b1_ref[...] + uhb_ref[...])
    mu_ref[...] = (_dot(a0, wmu_ref[0:128, :]) + _dot(a1, wmu_ref[128:256, :])
                   + bmu_ref[...])
    ls_ref[...] = (_dot(a0, wls_ref[0:128, :]) + _dot(a1, wls_ref[128:256, :])
                   + bls_ref[...])


def _tc3_call(s2a, s2b, uh, dinv, Wmu, bmur, Wls, blsr):
    half = lambda off: pl.BlockSpec((RB, OUT), lambda i, off=off: (off + i, 0))
    return pl.pallas_call(
        _tc3_body,
        grid=(NBLK,),
        in_specs=[
            half(0), half(NBLK),
            half(0), half(NBLK),
            half(0), half(NBLK),
            pl.BlockSpec((RB, 1), lambda i: (i, 0)),
            pl.BlockSpec((HID, OUT), lambda i: (0, 0)),
            pl.BlockSpec((1, OUT), lambda i: (0, 0)),
            pl.BlockSpec((HID, OUT), lambda i: (0, 0)),
            pl.BlockSpec((1, OUT), lambda i: (0, 0)),
        ],
        out_specs=[
            pl.BlockSpec((RB, OUT), lambda i: (i, 0)),
            pl.BlockSpec((RB, OUT), lambda i: (i, 0)),
        ],
        out_shape=[
            jax.ShapeDtypeStruct((N, OUT), _f32),
            jax.ShapeDtypeStruct((N, OUT), _f32),
        ],
    )(s2a, s2a, s2b, s2b, uh, uh, dinv, Wmu, bmur, Wls, blsr)


@jax.jit
def _run(x, edge_index, W1, b1, Wmu, bmu, Wls, bls):
    ei = edge_index.astype(jnp.int32)
    src, dst = ei[0], ei[1]
    dst1 = dst.reshape(NW, CD, BD)
    src5 = src.reshape(NW, NSEG, CSEG, B)
    dst5 = dst.reshape(NW, NSEG, CSEG, B)
    sd_a = jnp.stack([src5, dst5], axis=2)
    sd_b = jnp.stack([src5 + N, dst5], axis=2)
    sd_stack = jnp.stack([sd_a, sd_a, sd_b])
    z128 = jnp.zeros((RPT, 128), _f32)
    b1r = b1.reshape(1, HID)

    degt = _deg_call(dst1).T
    dinv, u = _tc1_call(degt, x)

    n_iter = lax.optimization_barrier(jnp.int32(3))

    def cond_fn(st):
        return st[0] < n_iter

    def body_fn(st):
        i, tab, _, s_prev = st
        sd_i = lax.dynamic_index_in_dim(sd_stack, i, keepdims=False)
        s = _agg_call(tab, sd_i, z128)
        new_tab = lax.cond(
            i == 0,
            lambda: _tc2_call(s, u, dinv, W1, b1r),
            lambda: tab,
        )
        return (i + 1, new_tab, s_prev, s)

    _, uh, s2a, s2b = lax.while_loop(
        cond_fn, body_fn, (jnp.int32(0), u, u, u))

    mu, ls = _tc3_call(s2a, s2b, uh, dinv, Wmu, bmu.reshape(1, OUT),
                       Wls, bls.reshape(1, OUT))
    return (mu, ls)


def kernel(x, edge_index, W1, b1, Wmu, bmu, Wls, bls):
    return _run(x, edge_index, W1, b1, Wmu, bmu, Wls, bls)
